# Initial kernel scaffold; baseline (speedup 1.0000x reference)
import jax
import jax.numpy as jnp
from jax import lax
from jax.experimental import pallas as pl
from jax.experimental.pallas import tpu as pltpu
from jax.experimental.pallas import tpu_sc as plsc

N = 10000
E = 320000
D = 128

K = 128
NUM_CORES = 2
NUM_SUBCORES = 16
NUM_WORKERS = NUM_CORES * NUM_SUBCORES
NCH_TOT = -(-E // K)
CH_PER_W = -(-NCH_TOT // NUM_WORKERS)
E_PAD = CH_PER_W * NUM_WORKERS * K
N_PAD = 10240
SLAB = N_PAD // NUM_SUBCORES
DUMMY_DST = N_PAD - 1


def _sc_agg_kernel(x_hbm, src_hbm, dst_hbm, zeros_hbm, out_hbm,
                   src_v, dst_v, rows_v, agg, sem):
    c = lax.axis_index("c")
    s = lax.axis_index("s")
    w = s * NUM_CORES + c

    pltpu.sync_copy(zeros_hbm, agg.at[pl.ds(s * SLAB, SLAB)])

    base = w * CH_PER_W
    pltpu.sync_copy(src_hbm.at[pl.ds(base, CH_PER_W)], src_v)
    pltpu.sync_copy(dst_hbm.at[pl.ds(base, CH_PER_W)], dst_v)

    plsc.subcore_barrier()

    def body(j, carry):
        pltpu.async_copy(x_hbm.at[src_v.at[j]], rows_v, sem).wait()
        pltpu.sync_copy(rows_v, agg.at[dst_v.at[j]], add=True)
        return carry

    lax.fori_loop(0, CH_PER_W, body, 0)

    plsc.subcore_barrier()

    pltpu.sync_copy(agg.at[pl.ds(s * SLAB, SLAB)],
                    out_hbm.at[c, pl.ds(s * SLAB, SLAB)])


@jax.jit
def _sc_aggregate(x, src2d, dst2d, zeros):
    mesh = plsc.VectorSubcoreMesh(core_axis_name="c", subcore_axis_name="s")
    return pl.kernel(
        _sc_agg_kernel,
        out_type=jax.ShapeDtypeStruct((NUM_CORES, N_PAD, D), jnp.float32),
        mesh=mesh,
        scratch_types=[
            pltpu.VMEM((CH_PER_W, K), jnp.int32),
            pltpu.VMEM((CH_PER_W, K), jnp.int32),
            pltpu.VMEM((K, D), jnp.float32),
            pltpu.VMEM_SHARED((N_PAD, D), jnp.float32),
            pltpu.SemaphoreType.DMA,
        ],
    )(x, src2d, dst2d, zeros)


def _tc_mlp_kernel(x_ref, parts_ref, w1_ref, b1_ref, w2_ref, b2_ref,
                   scale_ref, out_ref):
    h = x_ref[...] * scale_ref[0, 0] + parts_ref[0] + parts_ref[1]
    h = jnp.dot(h, w1_ref[...], preferred_element_type=jnp.float32)
    h = jnp.maximum(h + b1_ref[...], 0.0)
    o = jnp.dot(h, w2_ref[...], preferred_element_type=jnp.float32)
    out_ref[...] = jnp.maximum(o + b2_ref[...], 0.0)


_BLK = 2000


@jax.jit
def _tc_mlp(x, parts, W1, b1, W2, b2, scale):
    grid = N // _BLK
    return pl.pallas_call(
        _tc_mlp_kernel,
        grid=(grid,),
        in_specs=[
            pl.BlockSpec((_BLK, D), lambda i: (i, 0)),
            pl.BlockSpec((NUM_CORES, _BLK, D), lambda i: (0, i, 0)),
            pl.BlockSpec((D, D), lambda i: (0, 0)),
            pl.BlockSpec((1, D), lambda i: (0, 0)),
            pl.BlockSpec((D, D), lambda i: (0, 0)),
            pl.BlockSpec((1, D), lambda i: (0, 0)),
            pl.BlockSpec(memory_space=pltpu.SMEM),
        ],
        out_specs=pl.BlockSpec((_BLK, D), lambda i: (i, 0)),
        out_shape=jax.ShapeDtypeStruct((N, D), jnp.float32),
    )(x, parts, W1, b1, W2, b2, scale)


def kernel(x, edge_index, W1, b1, W2, b2, eps):
    src = edge_index[0]
    dst = edge_index[1]
    pad = E_PAD - E
    src2d = jnp.pad(src, (0, pad)).reshape(E_PAD // K, K)
    dst2d = jnp.pad(dst, (0, pad), constant_values=DUMMY_DST).reshape(
        E_PAD // K, K)
    zeros = jnp.zeros((SLAB, D), jnp.float32)
    parts = _sc_aggregate(x, src2d, dst2d, zeros)
    scale = (1.0 + eps).reshape(1, 1).astype(jnp.float32)
    return _tc_mlp(x, parts, W1, b1.reshape(1, D), W2, b2.reshape(1, D),
                   scale)


# speedup vs baseline: 3.0144x; 3.0144x over previous
import jax
import jax.numpy as jnp
from jax import lax
from jax.experimental import pallas as pl
from jax.experimental.pallas import tpu as pltpu
from jax.experimental.pallas import tpu_sc as plsc

N = 10000
E = 320000
D = 128

K = 128
NUM_CORES = 2
NUM_SUBCORES = 16
NUM_WORKERS = NUM_CORES * NUM_SUBCORES
NCH_TOT = -(-E // K)
CH_PER_W = 80
E_PAD = CH_PER_W * NUM_WORKERS * K
N_PAD = 10240
SLAB = N_PAD // NUM_SUBCORES
DUMMY_DST = N_PAD - 1


def _sc_agg_kernel(x_hbm, src_hbm, dst_hbm, zeros_hbm, out_hbm,
                   src_v, dst_v, rows_v, agg, sem):
    c = lax.axis_index("c")
    s = lax.axis_index("s")
    w = s * NUM_CORES + c

    pltpu.sync_copy(zeros_hbm, agg.at[pl.ds(s * SLAB, SLAB)])

    base = w * CH_PER_W
    pltpu.sync_copy(src_hbm.at[pl.ds(base, CH_PER_W)], src_v)
    pltpu.sync_copy(dst_hbm.at[pl.ds(base, CH_PER_W)], dst_v)

    plsc.subcore_barrier()

    def body(j, carry):
        pltpu.async_copy(x_hbm.at[src_v.at[j]], rows_v, sem).wait()
        pltpu.sync_copy(rows_v, agg.at[dst_v.at[j]], add=True)
        return carry

    lax.fori_loop(0, CH_PER_W, body, 0)

    plsc.subcore_barrier()

    pltpu.sync_copy(agg.at[pl.ds(s * SLAB, SLAB)],
                    out_hbm.at[c, pl.ds(s * SLAB, SLAB)])


@jax.jit
def _sc_aggregate(x, src2d, dst2d, zeros):
    mesh = plsc.VectorSubcoreMesh(core_axis_name="c", subcore_axis_name="s")
    return pl.kernel(
        _sc_agg_kernel,
        out_type=jax.ShapeDtypeStruct((NUM_CORES, N_PAD, D), jnp.float32),
        mesh=mesh,
        scratch_types=[
            pltpu.VMEM((CH_PER_W, K), jnp.int32),
            pltpu.VMEM((CH_PER_W, K), jnp.int32),
            pltpu.VMEM((K, D), jnp.float32),
            pltpu.VMEM_SHARED((N_PAD, D), jnp.float32),
            pltpu.SemaphoreType.DMA,
        ],
    )(x, src2d, dst2d, zeros)


def _tc_mlp_kernel(x_ref, parts_ref, w1_ref, b1_ref, w2_ref, b2_ref,
                   scale_ref, out_ref):
    h = x_ref[...] * scale_ref[0, 0] + parts_ref[0] + parts_ref[1]
    h = jnp.dot(h, w1_ref[...], preferred_element_type=jnp.float32)
    h = jnp.maximum(h + b1_ref[...], 0.0)
    o = jnp.dot(h, w2_ref[...], preferred_element_type=jnp.float32)
    out_ref[...] = jnp.maximum(o + b2_ref[...], 0.0)


_BLK = 2000


@jax.jit
def _tc_mlp(x, parts, W1, b1, W2, b2, scale):
    grid = N // _BLK
    return pl.pallas_call(
        _tc_mlp_kernel,
        grid=(grid,),
        in_specs=[
            pl.BlockSpec((_BLK, D), lambda i: (i, 0)),
            pl.BlockSpec((NUM_CORES, _BLK, D), lambda i: (0, i, 0)),
            pl.BlockSpec((D, D), lambda i: (0, 0)),
            pl.BlockSpec((1, D), lambda i: (0, 0)),
            pl.BlockSpec((D, D), lambda i: (0, 0)),
            pl.BlockSpec((1, D), lambda i: (0, 0)),
            pl.BlockSpec(memory_space=pltpu.SMEM),
        ],
        out_specs=pl.BlockSpec((_BLK, D), lambda i: (i, 0)),
        out_shape=jax.ShapeDtypeStruct((N, D), jnp.float32),
    )(x, parts, W1, b1, W2, b2, scale)


def kernel(x, edge_index, W1, b1, W2, b2, eps):
    src = edge_index[0]
    dst = edge_index[1]
    pad = E_PAD - E
    src2d = jnp.pad(src, (0, pad)).reshape(E_PAD // K, K)
    dst2d = jnp.pad(dst, (0, pad), constant_values=DUMMY_DST).reshape(
        E_PAD // K, K)
    zeros = jnp.zeros((SLAB, D), jnp.float32)
    parts = _sc_aggregate(x, src2d, dst2d, zeros)
    scale = (1.0 + eps).reshape(1, 1).astype(jnp.float32)
    return _tc_mlp(x, parts, W1, b1.reshape(1, D), W2, b2.reshape(1, D),
                   scale)
